# baseline (speedup 1.0000x reference)
import functools

import jax
import jax.numpy as jnp
from jax import lax
from jax.experimental import pallas as pl
from jax.experimental.pallas import tpu as pltpu
from jax.experimental.pallas import tpu_sc as plsc

_B = 16384
_B2 = 2 * _B
_D = 64
_MARGIN = 1.0
_NW = 32
_PER_W = _B2 // _NW
_CHUNK = 128
_NCHUNK = _PER_W // _CHUNK
_L = 16


def _sc_sqdist(h_idx, t_idx, r_idx, entity_emb, relation_emb, normal_emb):
    mesh = plsc.VectorSubcoreMesh(core_axis_name="c", subcore_axis_name="s")

    @functools.partial(
        pl.kernel,
        mesh=mesh,
        out_type=jax.ShapeDtypeStruct((_B2,), jnp.float32),
        compiler_params=pltpu.CompilerParams(
            use_tc_tiling_on_sc=False, needs_layout_passes=False),
        scratch_types=[
            pltpu.VMEM((_PER_W,), jnp.int32),
            pltpu.VMEM((_PER_W,), jnp.int32),
            pltpu.VMEM((_PER_W,), jnp.int32),
            pltpu.VMEM((_CHUNK, _D), jnp.float32),
            pltpu.VMEM((_CHUNK, _D), jnp.float32),
            pltpu.VMEM((_CHUNK, _D), jnp.float32),
            pltpu.VMEM((_CHUNK, _D), jnp.float32),
            pltpu.VMEM((_PER_W,), jnp.float32),
            pltpu.SemaphoreType.DMA,
        ],
    )
    def k(h_hbm, t_hbm, r_hbm, ent_hbm, rel_hbm, nrm_hbm, sq_hbm,
          hidx_v, tidx_v, ridx_v, rows_h, rows_t, rows_r, rows_w, out_v, sem):
        wid = lax.axis_index("s") * 2 + lax.axis_index("c")
        base = wid * _PER_W
        pltpu.sync_copy(h_hbm.at[pl.ds(base, _PER_W)], hidx_v)
        pltpu.sync_copy(t_hbm.at[pl.ds(base, _PER_W)], tidx_v)
        pltpu.sync_copy(r_hbm.at[pl.ds(base, _PER_W)], ridx_v)

        def chunk_body(ci, carry):
            off = ci * _CHUNK
            cp_h = pltpu.async_copy(
                ent_hbm.at[hidx_v.at[pl.ds(off, _CHUNK)]], rows_h, sem)
            cp_t = pltpu.async_copy(
                ent_hbm.at[tidx_v.at[pl.ds(off, _CHUNK)]], rows_t, sem)
            cp_r = pltpu.async_copy(
                rel_hbm.at[ridx_v.at[pl.ds(off, _CHUNK)]], rows_r, sem)
            cp_w = pltpu.async_copy(
                nrm_hbm.at[ridx_v.at[pl.ds(off, _CHUNK)]], rows_w, sem)
            cp_h.wait()
            cp_t.wait()
            cp_r.wait()
            cp_w.wait()

            def group_body(g, gcarry):
                row = g * _L + lax.iota(jnp.int32, _L)
                uu = jnp.zeros((_L,), jnp.float32)
                hw = jnp.zeros((_L,), jnp.float32)
                tw = jnp.zeros((_L,), jnp.float32)
                rw = jnp.zeros((_L,), jnp.float32)
                ww = jnp.zeros((_L,), jnp.float32)
                for d in range(_D):
                    col = jnp.full((_L,), d, jnp.int32)
                    hv = plsc.load_gather(rows_h, [row, col])
                    tv = plsc.load_gather(rows_t, [row, col])
                    rv = plsc.load_gather(rows_r, [row, col])
                    wv = plsc.load_gather(rows_w, [row, col])
                    uv = hv - tv + rv
                    uu = uu + uv * uv
                    hw = hw + hv * wv
                    tw = tw + tv * wv
                    rw = rw + rv * wv
                    ww = ww + wv * wv
                cdot = tw - hw
                sq = uu + 2.0 * cdot * rw + cdot * cdot * (ww - 2.0)
                out_v[pl.ds(off + g * _L, _L)] = sq
                return gcarry

            lax.fori_loop(0, _CHUNK // _L, group_body, 0, unroll=False)
            return carry

        lax.fori_loop(0, _NCHUNK, chunk_body, 0, unroll=False)
        pltpu.sync_copy(out_v, sq_hbm.at[pl.ds(base, _PER_W)])

    return k(h_idx, t_idx, r_idx, entity_emb, relation_emb, normal_emb)


def _finish_body(pos_ref, neg_ref, out_ref):
    pd = jnp.sqrt(pos_ref[...])
    nd = jnp.sqrt(neg_ref[...])
    out_ref[...] = jnp.sum(jnp.maximum(_MARGIN + pd - nd, 0.0)).reshape(1, 1)


def kernel(positive_triples, negative_triples, entity_emb, relation_emb, normal_emb):
    h_idx = jnp.concatenate([positive_triples[:, 0], negative_triples[:, 0]])
    t_idx = jnp.concatenate([positive_triples[:, 1], negative_triples[:, 1]])
    r_idx = jnp.concatenate([positive_triples[:, 2], negative_triples[:, 2]])

    sq = _sc_sqdist(h_idx, t_idx, r_idx, entity_emb, relation_emb, normal_emb)

    pos2 = sq[:_B].reshape(128, 128)
    neg2 = sq[_B:].reshape(128, 128)
    loss = pl.pallas_call(
        _finish_body,
        out_shape=jax.ShapeDtypeStruct((1, 1), jnp.float32),
    )(pos2, neg2)
    return loss[0, 0]
